# Initial kernel scaffold; baseline (speedup 1.0000x reference)
import jax
import jax.numpy as jnp
from jax.experimental import pallas as pl


def kernel(damage_type_ids, special_indices, numerical, damage_table, special_table):
    raise NotImplementedError("write your pallas kernel here")


# speedup vs baseline: 10.6982x; 10.6982x over previous
import functools

import jax
import jax.numpy as jnp
from jax import lax
from jax.experimental import pallas as pl
from jax.experimental.pallas import tpu as pltpu
from jax.experimental.pallas import tpu_sc as plsc

B = 16384
S = 20
DV = 6
DD = 16
SV = 12
SD = 16
NUM = 3
OUT = DD + SD + NUM

NC = 2
NS = 16
NW = NC * NS
BW = B // NW
L = 16
NG = BW // L


def _splat_i32(v):
    return jnp.full((L,), v, jnp.int32)


def _body(dmg_hbm, spec_hbm, num_hbm, dtab_hbm, stab_hbm, out_hbm,
          dmg_v, spec_v, num_v, dtab_v, stab_v, out_v):
    cid = lax.axis_index("c")
    sid = lax.axis_index("s")
    wid = sid * NC + cid
    base = wid * BW

    pltpu.sync_copy(dmg_hbm.at[pl.ds(base, BW)], dmg_v)
    pltpu.sync_copy(spec_hbm.at[pl.ds(base, BW)], spec_v)
    pltpu.sync_copy(num_hbm.at[pl.ds(base, BW)], num_v)
    pltpu.sync_copy(dtab_hbm, dtab_v)
    pltpu.sync_copy(stab_hbm, stab_v)

    lane = lax.iota(jnp.int32, L)
    inv_s = jnp.full((L,), 1.0 / S, jnp.float32)

    def group(g, carry):
        bidx = g * L + lane
        dv = dmg_v[pl.ds(g * L, L)]
        for d in range(DD):
            vals = plsc.load_gather(dtab_v, [dv, _splat_i32(d)])
            plsc.store_scatter(out_v, [bidx, _splat_i32(d)], vals)
        sidx = [plsc.load_gather(spec_v, [bidx, _splat_i32(s)])
                for s in range(S)]
        for d in range(SD):
            col = _splat_i32(d)
            acc = plsc.load_gather(stab_v, [sidx[0], col])
            for s in range(1, S):
                acc = acc + plsc.load_gather(stab_v, [sidx[s], col])
            plsc.store_scatter(out_v, [bidx, _splat_i32(DD + d)], acc * inv_s)
        for j in range(NUM):
            vals = plsc.load_gather(num_v, [bidx, _splat_i32(j)])
            plsc.store_scatter(out_v, [bidx, _splat_i32(DD + SD + j)], vals)
        return carry

    lax.fori_loop(0, NG, group, 0)

    pltpu.sync_copy(out_v, out_hbm.at[pl.ds(base, BW)])


@jax.jit
def _encode(damage_type_ids, special_indices, numerical, damage_table,
            special_table):
    mesh = plsc.VectorSubcoreMesh(core_axis_name="c", subcore_axis_name="s")
    run = functools.partial(
        pl.kernel,
        mesh=mesh,
        out_type=jax.ShapeDtypeStruct((B, OUT), jnp.float32),
        compiler_params=pltpu.CompilerParams(needs_layout_passes=False,
                                             use_tc_tiling_on_sc=False),
        scratch_types=[
            pltpu.VMEM((BW,), jnp.int32),
            pltpu.VMEM((BW, S), jnp.int32),
            pltpu.VMEM((BW, NUM), jnp.float32),
            pltpu.VMEM((DV, DD), jnp.float32),
            pltpu.VMEM((SV, SD), jnp.float32),
            pltpu.VMEM((BW, OUT), jnp.float32),
        ],
    )(_body)
    return run(damage_type_ids, special_indices, numerical, damage_table,
               special_table)


def kernel(damage_type_ids, special_indices, numerical, damage_table,
           special_table):
    return _encode(damage_type_ids.astype(jnp.int32),
                   special_indices.astype(jnp.int32),
                   numerical, damage_table, special_table)


# speedup vs baseline: 12.7497x; 1.1918x over previous
import functools

import jax
import jax.numpy as jnp
from jax import lax
from jax.experimental import pallas as pl
from jax.experimental.pallas import tpu as pltpu
from jax.experimental.pallas import tpu_sc as plsc

B = 16384
S = 20
DV = 6
DD = 16
SV = 12
SD = 16
NUM = 3
OUT = DD + SD + NUM

NC = 2
NS = 16
NW = NC * NS
BW = B // NW
L = 16
NG = BW // L
PV = SV * SV


def _splat_i32(v):
    return jnp.full((L,), v, jnp.int32)


def _body(dmg_hbm, spec_hbm, num_hbm, dtab_hbm, stab_hbm, out_hbm,
          dmg_v, spec_v, num_v, dtab_v, stab_v, out_v, ptab_v):
    cid = lax.axis_index("c")
    sid = lax.axis_index("s")
    wid = sid * NC + cid
    base = wid * BW

    pltpu.sync_copy(dmg_hbm.at[pl.ds(base, BW)], dmg_v)
    pltpu.sync_copy(spec_hbm.at[pl.ds(base, BW)], spec_v)
    pltpu.sync_copy(num_hbm.at[pl.ds(base, BW)], num_v)
    pltpu.sync_copy(dtab_hbm, dtab_v)
    pltpu.sync_copy(stab_hbm, stab_v)

    lane = lax.iota(jnp.int32, L)
    inv_s = jnp.full((L,), 1.0 / S, jnp.float32)

    for pg in range(PV // L):
        pvec = pg * L + lane
        pa = pvec // SV
        pb = pvec - pa * SV
        for d in range(SD):
            col = _splat_i32(d)
            va = plsc.load_gather(stab_v, [pa, col])
            vb = plsc.load_gather(stab_v, [pb, col])
            plsc.store_scatter(ptab_v, [pvec, col], va + vb)

    def _tree_sum(vals):
        while len(vals) > 1:
            vals = [a + b for a, b in zip(vals[::2], vals[1::2])] + (
                [vals[-1]] if len(vals) % 2 else [])
        return vals[0]

    def group(g, carry):
        bidx = g * L + lane
        dv = dmg_v[pl.ds(g * L, L)]
        for d in range(DD):
            vals = plsc.load_gather(dtab_v, [dv, _splat_i32(d)])
            plsc.store_scatter(out_v, [bidx, _splat_i32(d)], vals)
        sidx = [plsc.load_gather(spec_v, [bidx, _splat_i32(s)])
                for s in range(S)]
        pidx = [sidx[2 * t] * SV + sidx[2 * t + 1] for t in range(S // 2)]
        for d in range(SD):
            col = _splat_i32(d)
            acc = _tree_sum([plsc.load_gather(ptab_v, [p, col])
                             for p in pidx])
            plsc.store_scatter(out_v, [bidx, _splat_i32(DD + d)], acc * inv_s)
        for j in range(NUM):
            vals = plsc.load_gather(num_v, [bidx, _splat_i32(j)])
            plsc.store_scatter(out_v, [bidx, _splat_i32(DD + SD + j)], vals)
        return carry

    lax.fori_loop(0, NG, group, 0)

    pltpu.sync_copy(out_v, out_hbm.at[pl.ds(base, BW)])


@jax.jit
def _encode(damage_type_ids, special_indices, numerical, damage_table,
            special_table):
    mesh = plsc.VectorSubcoreMesh(core_axis_name="c", subcore_axis_name="s")
    run = functools.partial(
        pl.kernel,
        mesh=mesh,
        out_type=jax.ShapeDtypeStruct((B, OUT), jnp.float32),
        compiler_params=pltpu.CompilerParams(needs_layout_passes=False,
                                             use_tc_tiling_on_sc=False),
        scratch_types=[
            pltpu.VMEM((BW,), jnp.int32),
            pltpu.VMEM((BW, S), jnp.int32),
            pltpu.VMEM((BW, NUM), jnp.float32),
            pltpu.VMEM((DV, DD), jnp.float32),
            pltpu.VMEM((SV, SD), jnp.float32),
            pltpu.VMEM((BW, OUT), jnp.float32),
            pltpu.VMEM((PV, SD), jnp.float32),
        ],
    )(_body)
    return run(damage_type_ids, special_indices, numerical, damage_table,
               special_table)


def kernel(damage_type_ids, special_indices, numerical, damage_table,
           special_table):
    return _encode(damage_type_ids.astype(jnp.int32),
                   special_indices.astype(jnp.int32),
                   numerical, damage_table, special_table)


# speedup vs baseline: 17.3412x; 1.3601x over previous
import functools

import jax
import jax.numpy as jnp
from jax import lax
from jax.experimental import pallas as pl
from jax.experimental.pallas import tpu as pltpu
from jax.experimental.pallas import tpu_sc as plsc

B = 16384
S = 20
DV = 6
DD = 16
SV = 12
SD = 16
NUM = 3
OUT = DD + SD + NUM
TP = 17

NC = 2
NS = 16
NW = NC * NS
BW = B // NW
L = 16
NG = BW // L
PV = SV * SV


def _splat_i32(v):
    return jnp.full((L,), v, jnp.int32)


def _tree_sum(vals):
    while len(vals) > 1:
        vals = [a + b for a, b in zip(vals[::2], vals[1::2])] + (
            [vals[-1]] if len(vals) % 2 else [])
    return vals[0]


def _body(dmg_hbm, spec_hbm, num_hbm, dtab_hbm, stab_hbm, out_hbm,
          dmg_v, spec_v, num_v, dtab_v, stab_v, out_v, ptab_v):
    cid = lax.axis_index("c")
    sid = lax.axis_index("s")
    wid = sid * NC + cid
    base = wid * BW

    pltpu.sync_copy(dmg_hbm.at[pl.ds(base, BW)], dmg_v)
    pltpu.sync_copy(spec_hbm.at[:, pl.ds(base, BW)], spec_v)
    pltpu.sync_copy(num_hbm.at[pl.ds(base, BW)], num_v)
    pltpu.sync_copy(dtab_hbm, dtab_v)
    pltpu.sync_copy(stab_hbm, stab_v)

    lane = lax.iota(jnp.int32, L)
    inv_s = jnp.full((L,), 1.0 / S, jnp.float32)

    for pg in range(PV // L):
        pvec = pg * L + lane
        pa = pvec // SV
        pb = pvec - pa * SV
        for d in range(SD):
            col = _splat_i32(d)
            va = plsc.load_gather(stab_v, [pa, col])
            vb = plsc.load_gather(stab_v, [pb, col])
            plsc.store_scatter(ptab_v, [pvec, col], va + vb)

    def group(g, carry):
        bidx = g * L + lane
        dv = dmg_v[pl.ds(g * L, L)]
        for d in range(DD):
            vals = plsc.load_gather(dtab_v, [dv, _splat_i32(d)])
            plsc.store_scatter(out_v, [bidx, _splat_i32(d)], vals)
        sidx = [spec_v[s, pl.ds(g * L, L)] for s in range(S)]
        pidx = [sidx[2 * t] * SV + sidx[2 * t + 1] for t in range(S // 2)]
        for d in range(SD):
            col = _splat_i32(d)
            acc = _tree_sum([plsc.load_gather(ptab_v, [p, col])
                             for p in pidx])
            plsc.store_scatter(out_v, [bidx, _splat_i32(DD + d)], acc * inv_s)
        for j in range(NUM):
            vals = plsc.load_gather(num_v, [bidx, _splat_i32(j)])
            plsc.store_scatter(out_v, [bidx, _splat_i32(DD + SD + j)], vals)
        return carry

    lax.fori_loop(0, NG, group, 0)

    pltpu.sync_copy(out_v, out_hbm.at[pl.ds(base, BW)])


@jax.jit
def _encode(damage_type_ids, special_indices_t, numerical, damage_table_p,
            special_table_p):
    mesh = plsc.VectorSubcoreMesh(core_axis_name="c", subcore_axis_name="s")
    run = functools.partial(
        pl.kernel,
        mesh=mesh,
        out_type=jax.ShapeDtypeStruct((B, OUT), jnp.float32),
        compiler_params=pltpu.CompilerParams(needs_layout_passes=False,
                                             use_tc_tiling_on_sc=False),
        scratch_types=[
            pltpu.VMEM((BW,), jnp.int32),
            pltpu.VMEM((S, BW), jnp.int32),
            pltpu.VMEM((BW, NUM), jnp.float32),
            pltpu.VMEM((DV, TP), jnp.float32),
            pltpu.VMEM((SV, TP), jnp.float32),
            pltpu.VMEM((BW, OUT), jnp.float32),
            pltpu.VMEM((PV, TP), jnp.float32),
        ],
    )(_body)
    return run(damage_type_ids, special_indices_t, numerical, damage_table_p,
               special_table_p)


def kernel(damage_type_ids, special_indices, numerical, damage_table,
           special_table):
    pad = [(0, 0), (0, TP - DD)]
    return _encode(damage_type_ids.astype(jnp.int32),
                   special_indices.astype(jnp.int32).T,
                   numerical,
                   jnp.pad(damage_table, pad),
                   jnp.pad(special_table, pad))
